# baseline (speedup 1.0000x reference)
import jax
import jax.numpy as jnp
from jax import lax
from jax.experimental import pallas as pl
from jax.experimental.pallas import tpu as pltpu
from jax.experimental.pallas import tpu_sc as plsc

_VOCAB = 1000000
_DIM = 64
_BATCH = 16384

_INFO = plsc.get_sparse_core_info()
_NC = _INFO.num_cores
_NS = _INFO.num_subcores
_L = _INFO.num_lanes
_NW = _NC * _NS
_BPW = _BATCH // _NW
_CHUNK = 128
_NCHUNK = _BPW // _CHUNK
_PD = 2 * _DIM
_NBLK = 7813
_PROWS = _NBLK * _DIM


def _conv_kernel(t_ref, c_ref, to_ref, co_ref):
    for ref, oref in ((t_ref, to_ref), (c_ref, co_ref)):
        tt = jnp.transpose(ref[...])
        oref[:, 0:_DIM] = tt[0:_DIM, :]
        oref[:, _DIM:_PD] = tt[_DIM:2 * _DIM, :]


def _dot_kernel(t_idx, c_idx, t_tab, c_tab, out_hbm,
                tidx_v, cidx_v, tpair_v, cpair_v, tb, cb, out_v,
                tsem0, tsem1, csem0, csem1):
    wid = lax.axis_index("s") * _NC + lax.axis_index("c")
    base = wid * _BPW

    pltpu.sync_copy(t_idx.at[pl.ds(base, _BPW)], tidx_v)
    pltpu.sync_copy(c_idx.at[pl.ds(base, _BPW)], cidx_v)

    def pair_body(i, carry):
        sl = pl.ds(i * _L, _L)
        tv = tidx_v[sl]
        cv = cidx_v[sl]
        tpair_v[sl] = ((tv >> 7) << 6) + (tv & 63)
        cpair_v[sl] = ((cv >> 7) << 6) + (cv & 63)
        return carry
    lax.fori_loop(0, _BPW // _L, pair_body, 0)

    tsems = (tsem0, tsem1)
    csems = (csem0, csem1)

    def fire(j):
        slot = j % 2
        sl = pl.ds(j * _CHUNK, _CHUNK)
        tcp = pltpu.make_async_copy(t_tab.at[tpair_v.at[sl]], tb.at[slot], tsems[slot])
        ccp = pltpu.make_async_copy(c_tab.at[cpair_v.at[sl]], cb.at[slot], csems[slot])
        tcp.start()
        ccp.start()
        return tcp, ccp

    lane = lax.iota(jnp.int32, 16)

    def compute_chunk(j):
        slot = j % 2
        trows = tb.at[slot]
        crows = cb.at[slot]

        def group_body(g, carry):
            sl = pl.ds(j * _CHUNK + g * _L, _L)
            toff = ((tidx_v[sl] >> 6) & 1) * _DIM
            coff = ((cidx_v[sl] >> 6) & 1) * _DIM
            rows = lane + g * _L

            def d_body(d, acc):
                tv = plsc.load_gather(trows, [rows, toff + d])
                cv = plsc.load_gather(crows, [rows, coff + d])
                return acc + tv * cv

            acc = lax.fori_loop(0, _DIM, d_body, jnp.zeros((16,), jnp.float32))
            out_v[pl.ds(j * _CHUNK + g * _L, _L)] = acc
            return carry

        lax.fori_loop(0, _CHUNK // _L, group_body, 0)

    pending = fire(0)
    for j in range(_NCHUNK):
        nxt = fire(j + 1) if j + 1 < _NCHUNK else None
        for cp in pending:
            cp.wait()
        compute_chunk(j)
        pending = nxt

    pltpu.sync_copy(out_v, out_hbm.at[pl.ds(base, _BPW)])


@jax.jit
def _run(target, context, target_table, context_table):
    t = target.astype(jnp.int32)
    c = context.astype(jnp.int32)
    t3 = jnp.transpose(target_table)
    c3 = jnp.transpose(context_table)

    tconv, cconv = pl.pallas_call(
        _conv_kernel,
        grid=(_NBLK,),
        in_specs=[
            pl.BlockSpec((_DIM, _PD), lambda i: (0, i)),
            pl.BlockSpec((_DIM, _PD), lambda i: (0, i)),
        ],
        out_specs=[
            pl.BlockSpec((_DIM, _PD), lambda i: (i, 0)),
            pl.BlockSpec((_DIM, _PD), lambda i: (i, 0)),
        ],
        out_shape=[
            jax.ShapeDtypeStruct((_PROWS, _PD), jnp.float32),
            jax.ShapeDtypeStruct((_PROWS, _PD), jnp.float32),
        ],
        compiler_params=pltpu.CompilerParams(
            dimension_semantics=("arbitrary",),
        ),
    )(t3, c3)

    mesh = plsc.VectorSubcoreMesh(core_axis_name="c", subcore_axis_name="s")
    k = pl.kernel(
        _dot_kernel,
        out_type=jax.ShapeDtypeStruct((_BATCH,), jnp.float32),
        mesh=mesh,
        scratch_types=[
            pltpu.VMEM((_BPW,), jnp.int32),
            pltpu.VMEM((_BPW,), jnp.int32),
            pltpu.VMEM((_BPW,), jnp.int32),
            pltpu.VMEM((_BPW,), jnp.int32),
            pltpu.VMEM((2, _CHUNK, _PD), jnp.float32),
            pltpu.VMEM((2, _CHUNK, _PD), jnp.float32),
            pltpu.VMEM((_BPW,), jnp.float32),
            pltpu.SemaphoreType.DMA,
            pltpu.SemaphoreType.DMA,
            pltpu.SemaphoreType.DMA,
            pltpu.SemaphoreType.DMA,
        ],
        compiler_params=pltpu.CompilerParams(
            needs_layout_passes=False,
        ),
    )
    return k(t, c, tconv, cconv).reshape(_BATCH, 1)


def kernel(target, context, target_table, context_table):
    return _run(target, context, target_table, context_table)


# speedup vs baseline: 6.4945x; 6.4945x over previous
import jax
import jax.numpy as jnp
from jax import lax
from jax.experimental import pallas as pl
from jax.experimental.pallas import tpu as pltpu
from jax.experimental.pallas import tpu_sc as plsc

_VOCAB = 1000000
_DIM = 64
_BATCH = 16384

_INFO = plsc.get_sparse_core_info()
_NC = _INFO.num_cores
_NS = _INFO.num_subcores
_L = _INFO.num_lanes
_NW = _NC * _NS
_BPW = _BATCH // _NW
_CHUNK = 128
_NCHUNK = _BPW // _CHUNK
_PD = 2 * _DIM
_KSUB = 16
_NBLK = 489
_PROWS = _NBLK * _KSUB * _DIM


def _conv_kernel(t_ref, c_ref, to_ref, co_ref):
    for ref, oref in ((t_ref, to_ref), (c_ref, co_ref)):
        tt = jnp.transpose(ref[...])
        for k in range(_KSUB):
            oref[k * _DIM:(k + 1) * _DIM, 0:_DIM] = tt[k * _PD:k * _PD + _DIM, :]
            oref[k * _DIM:(k + 1) * _DIM, _DIM:_PD] = tt[k * _PD + _DIM:(k + 1) * _PD, :]


def _dot_kernel(t_idx, c_idx, t_tab, c_tab, out_hbm,
                tidx_v, cidx_v, tpair_v, cpair_v, tb, cb, out_v,
                tsem0, tsem1, csem0, csem1):
    wid = lax.axis_index("s") * _NC + lax.axis_index("c")
    base = wid * _BPW

    pltpu.sync_copy(t_idx.at[pl.ds(base, _BPW)], tidx_v)
    pltpu.sync_copy(c_idx.at[pl.ds(base, _BPW)], cidx_v)

    def pair_body(i, carry):
        sl = pl.ds(i * _L, _L)
        tv = tidx_v[sl]
        cv = cidx_v[sl]
        tpair_v[sl] = ((tv >> 7) << 6) + (tv & 63)
        cpair_v[sl] = ((cv >> 7) << 6) + (cv & 63)
        return carry
    lax.fori_loop(0, _BPW // _L, pair_body, 0)

    tsems = (tsem0, tsem1)
    csems = (csem0, csem1)

    def fire(j):
        slot = j % 2
        sl = pl.ds(j * _CHUNK, _CHUNK)
        tcp = pltpu.make_async_copy(t_tab.at[tpair_v.at[sl]], tb.at[slot], tsems[slot])
        ccp = pltpu.make_async_copy(c_tab.at[cpair_v.at[sl]], cb.at[slot], csems[slot])
        tcp.start()
        ccp.start()
        return tcp, ccp

    lane = lax.iota(jnp.int32, 16)

    def compute_chunk(j):
        slot = j % 2
        trows = tb.at[slot]
        crows = cb.at[slot]

        def group_body(g, carry):
            sl = pl.ds(j * _CHUNK + g * _L, _L)
            toff = ((tidx_v[sl] >> 6) & 1) * _DIM
            coff = ((cidx_v[sl] >> 6) & 1) * _DIM
            rows = lane + g * _L

            def d_body(d, acc):
                tv = plsc.load_gather(trows, [rows, toff + d])
                cv = plsc.load_gather(crows, [rows, coff + d])
                return acc + tv * cv

            acc = lax.fori_loop(0, _DIM, d_body, jnp.zeros((16,), jnp.float32))
            out_v[pl.ds(j * _CHUNK + g * _L, _L)] = acc
            return carry

        lax.fori_loop(0, _CHUNK // _L, group_body, 0)

    pending = fire(0)
    for j in range(_NCHUNK):
        nxt = fire(j + 1) if j + 1 < _NCHUNK else None
        for cp in pending:
            cp.wait()
        compute_chunk(j)
        pending = nxt

    pltpu.sync_copy(out_v, out_hbm.at[pl.ds(base, _BPW)])


@jax.jit
def _run(target, context, target_table, context_table):
    t = target.astype(jnp.int32)
    c = context.astype(jnp.int32)
    t3 = jnp.transpose(target_table)
    c3 = jnp.transpose(context_table)

    tconv, cconv = pl.pallas_call(
        _conv_kernel,
        grid=(_NBLK,),
        in_specs=[
            pl.BlockSpec((_DIM, _KSUB * _PD), lambda i: (0, i)),
            pl.BlockSpec((_DIM, _KSUB * _PD), lambda i: (0, i)),
        ],
        out_specs=[
            pl.BlockSpec((_KSUB * _DIM, _PD), lambda i: (i, 0)),
            pl.BlockSpec((_KSUB * _DIM, _PD), lambda i: (i, 0)),
        ],
        out_shape=[
            jax.ShapeDtypeStruct((_PROWS, _PD), jnp.float32),
            jax.ShapeDtypeStruct((_PROWS, _PD), jnp.float32),
        ],
        compiler_params=pltpu.CompilerParams(
            dimension_semantics=("arbitrary",),
        ),
    )(t3, c3)

    mesh = plsc.VectorSubcoreMesh(core_axis_name="c", subcore_axis_name="s")
    k = pl.kernel(
        _dot_kernel,
        out_type=jax.ShapeDtypeStruct((_BATCH,), jnp.float32),
        mesh=mesh,
        scratch_types=[
            pltpu.VMEM((_BPW,), jnp.int32),
            pltpu.VMEM((_BPW,), jnp.int32),
            pltpu.VMEM((_BPW,), jnp.int32),
            pltpu.VMEM((_BPW,), jnp.int32),
            pltpu.VMEM((2, _CHUNK, _PD), jnp.float32),
            pltpu.VMEM((2, _CHUNK, _PD), jnp.float32),
            pltpu.VMEM((_BPW,), jnp.float32),
            pltpu.SemaphoreType.DMA,
            pltpu.SemaphoreType.DMA,
            pltpu.SemaphoreType.DMA,
            pltpu.SemaphoreType.DMA,
        ],
        compiler_params=pltpu.CompilerParams(
            needs_layout_passes=False,
        ),
    )
    return k(t, c, tconv, cconv).reshape(_BATCH, 1)


def kernel(target, context, target_table, context_table):
    return _run(target, context, target_table, context_table)


# speedup vs baseline: 9.5604x; 1.4721x over previous
import jax
import jax.numpy as jnp
from jax import lax
from jax.experimental import pallas as pl
from jax.experimental.pallas import tpu as pltpu
from jax.experimental.pallas import tpu_sc as plsc

_VOCAB = 1000000
_DIM = 64
_BATCH = 16384

_INFO = plsc.get_sparse_core_info()
_NC = _INFO.num_cores
_NS = _INFO.num_subcores
_L = _INFO.num_lanes
_NW = _NC * _NS
_BPW = _BATCH // _NW
_CHUNK = 128
_NCHUNK = _BPW // _CHUNK
_PD = 2 * _DIM
_KSUB = 64
_NBLK = 123
_PROWS = _NBLK * _KSUB * _DIM


def _conv_kernel(t_ref, c_ref, to_ref, co_ref):
    for ref, oref in ((t_ref, to_ref), (c_ref, co_ref)):
        tt = jnp.transpose(ref[...])
        for k in range(_KSUB):
            oref[k * _DIM:(k + 1) * _DIM, 0:_DIM] = tt[k * _PD:k * _PD + _DIM, :]
            oref[k * _DIM:(k + 1) * _DIM, _DIM:_PD] = tt[k * _PD + _DIM:(k + 1) * _PD, :]


def _dot_kernel(t_idx, c_idx, t_tab, c_tab, out_hbm,
                tidx_v, cidx_v, tpair_v, cpair_v, tb, cb, out_v,
                tsem0, tsem1, csem0, csem1):
    wid = lax.axis_index("s") * _NC + lax.axis_index("c")
    base = wid * _BPW

    pltpu.sync_copy(t_idx.at[pl.ds(base, _BPW)], tidx_v)
    pltpu.sync_copy(c_idx.at[pl.ds(base, _BPW)], cidx_v)

    def pair_body(i, carry):
        sl = pl.ds(i * _L, _L)
        tv = tidx_v[sl]
        cv = cidx_v[sl]
        tpair_v[sl] = ((tv >> 7) << 6) + (tv & 63)
        cpair_v[sl] = ((cv >> 7) << 6) + (cv & 63)
        return carry
    lax.fori_loop(0, _BPW // _L, pair_body, 0)

    tsems = (tsem0, tsem1)
    csems = (csem0, csem1)

    def fire(j):
        slot = j % 2
        sl = pl.ds(j * _CHUNK, _CHUNK)
        tcp = pltpu.make_async_copy(t_tab.at[tpair_v.at[sl]], tb.at[slot], tsems[slot])
        ccp = pltpu.make_async_copy(c_tab.at[cpair_v.at[sl]], cb.at[slot], csems[slot])
        tcp.start()
        ccp.start()
        return tcp, ccp

    lane = lax.iota(jnp.int32, 16)

    def compute_chunk(j):
        slot = j % 2
        trows = tb.at[slot]
        crows = cb.at[slot]

        def group_body(g, carry):
            sl = pl.ds(j * _CHUNK + g * _L, _L)
            toff = ((tidx_v[sl] >> 6) & 1) * _DIM
            coff = ((cidx_v[sl] >> 6) & 1) * _DIM
            rows = lane + g * _L

            def d_body(d, acc):
                tv = plsc.load_gather(trows, [rows, toff + d])
                cv = plsc.load_gather(crows, [rows, coff + d])
                return acc + tv * cv

            acc = lax.fori_loop(0, _DIM, d_body, jnp.zeros((16,), jnp.float32))
            out_v[pl.ds(j * _CHUNK + g * _L, _L)] = acc
            return carry

        lax.fori_loop(0, _CHUNK // _L, group_body, 0)

    pending = fire(0)
    for j in range(_NCHUNK):
        nxt = fire(j + 1) if j + 1 < _NCHUNK else None
        for cp in pending:
            cp.wait()
        compute_chunk(j)
        pending = nxt

    pltpu.sync_copy(out_v, out_hbm.at[pl.ds(base, _BPW)])


@jax.jit
def _run(target, context, target_table, context_table):
    t = target.astype(jnp.int32)
    c = context.astype(jnp.int32)
    t3 = jnp.transpose(target_table)
    c3 = jnp.transpose(context_table)

    tconv, cconv = pl.pallas_call(
        _conv_kernel,
        grid=(_NBLK,),
        in_specs=[
            pl.BlockSpec((_DIM, _KSUB * _PD), lambda i: (0, i)),
            pl.BlockSpec((_DIM, _KSUB * _PD), lambda i: (0, i)),
        ],
        out_specs=[
            pl.BlockSpec((_KSUB * _DIM, _PD), lambda i: (i, 0)),
            pl.BlockSpec((_KSUB * _DIM, _PD), lambda i: (i, 0)),
        ],
        out_shape=[
            jax.ShapeDtypeStruct((_PROWS, _PD), jnp.float32),
            jax.ShapeDtypeStruct((_PROWS, _PD), jnp.float32),
        ],
        compiler_params=pltpu.CompilerParams(
            dimension_semantics=("arbitrary",),
        ),
    )(t3, c3)

    mesh = plsc.VectorSubcoreMesh(core_axis_name="c", subcore_axis_name="s")
    k = pl.kernel(
        _dot_kernel,
        out_type=jax.ShapeDtypeStruct((_BATCH,), jnp.float32),
        mesh=mesh,
        scratch_types=[
            pltpu.VMEM((_BPW,), jnp.int32),
            pltpu.VMEM((_BPW,), jnp.int32),
            pltpu.VMEM((_BPW,), jnp.int32),
            pltpu.VMEM((_BPW,), jnp.int32),
            pltpu.VMEM((2, _CHUNK, _PD), jnp.float32),
            pltpu.VMEM((2, _CHUNK, _PD), jnp.float32),
            pltpu.VMEM((_BPW,), jnp.float32),
            pltpu.SemaphoreType.DMA,
            pltpu.SemaphoreType.DMA,
            pltpu.SemaphoreType.DMA,
            pltpu.SemaphoreType.DMA,
        ],
        compiler_params=pltpu.CompilerParams(
            needs_layout_passes=False,
        ),
    )
    return k(t, c, tconv, cconv).reshape(_BATCH, 1)


def kernel(target, context, target_table, context_table):
    return _run(target, context, target_table, context_table)


# speedup vs baseline: 9.6875x; 1.0133x over previous
import jax
import jax.numpy as jnp
from jax import lax
from jax.experimental import pallas as pl
from jax.experimental.pallas import tpu as pltpu
from jax.experimental.pallas import tpu_sc as plsc

_VOCAB = 1000000
_DIM = 64
_BATCH = 16384

_INFO = plsc.get_sparse_core_info()
_NC = _INFO.num_cores
_NS = _INFO.num_subcores
_L = _INFO.num_lanes
_NW = _NC * _NS
_BPW = _BATCH // _NW
_CHUNK = 128
_NCHUNK = _BPW // _CHUNK
_PD = 2 * _DIM
_KSUB = 128
_NBLK = 62
_PROWS = _NBLK * _KSUB * _DIM


def _conv_kernel(t_ref, c_ref, to_ref, co_ref):
    for ref, oref in ((t_ref, to_ref), (c_ref, co_ref)):
        tt = jnp.transpose(ref[...])
        for k in range(_KSUB):
            oref[k * _DIM:(k + 1) * _DIM, 0:_DIM] = tt[k * _PD:k * _PD + _DIM, :]
            oref[k * _DIM:(k + 1) * _DIM, _DIM:_PD] = tt[k * _PD + _DIM:(k + 1) * _PD, :]


def _dot_kernel(t_idx, c_idx, t_tab, c_tab, out_hbm,
                tidx_v, cidx_v, tpair_v, cpair_v, tb, cb, out_v,
                tsem0, tsem1, csem0, csem1):
    wid = lax.axis_index("s") * _NC + lax.axis_index("c")
    base = wid * _BPW

    pltpu.sync_copy(t_idx.at[pl.ds(base, _BPW)], tidx_v)
    pltpu.sync_copy(c_idx.at[pl.ds(base, _BPW)], cidx_v)

    def pair_body(i, carry):
        sl = pl.ds(i * _L, _L)
        tv = tidx_v[sl]
        cv = cidx_v[sl]
        tpair_v[sl] = ((tv >> 7) << 6) + (tv & 63)
        cpair_v[sl] = ((cv >> 7) << 6) + (cv & 63)
        return carry
    lax.fori_loop(0, _BPW // _L, pair_body, 0)

    tsems = (tsem0, tsem1)
    csems = (csem0, csem1)

    def fire(j):
        slot = j % 2
        sl = pl.ds(j * _CHUNK, _CHUNK)
        tcp = pltpu.make_async_copy(t_tab.at[tpair_v.at[sl]], tb.at[slot], tsems[slot])
        ccp = pltpu.make_async_copy(c_tab.at[cpair_v.at[sl]], cb.at[slot], csems[slot])
        tcp.start()
        ccp.start()
        return tcp, ccp

    lane = lax.iota(jnp.int32, 16)

    def compute_chunk(j):
        slot = j % 2
        trows = tb.at[slot]
        crows = cb.at[slot]

        def group_body(g, carry):
            sl = pl.ds(j * _CHUNK + g * _L, _L)
            toff = ((tidx_v[sl] >> 6) & 1) * _DIM
            coff = ((cidx_v[sl] >> 6) & 1) * _DIM
            rows = lane + g * _L

            def d_body(d, acc):
                tv = plsc.load_gather(trows, [rows, toff + d])
                cv = plsc.load_gather(crows, [rows, coff + d])
                return acc + tv * cv

            acc = lax.fori_loop(0, _DIM, d_body, jnp.zeros((16,), jnp.float32))
            out_v[pl.ds(j * _CHUNK + g * _L, _L)] = acc
            return carry

        lax.fori_loop(0, _CHUNK // _L, group_body, 0)

    pending = fire(0)
    for j in range(_NCHUNK):
        nxt = fire(j + 1) if j + 1 < _NCHUNK else None
        for cp in pending:
            cp.wait()
        compute_chunk(j)
        pending = nxt

    pltpu.sync_copy(out_v, out_hbm.at[pl.ds(base, _BPW)])


@jax.jit
def _run(target, context, target_table, context_table):
    t = target.astype(jnp.int32)
    c = context.astype(jnp.int32)
    t3 = jnp.transpose(target_table)
    c3 = jnp.transpose(context_table)

    tconv, cconv = pl.pallas_call(
        _conv_kernel,
        grid=(_NBLK,),
        in_specs=[
            pl.BlockSpec((_DIM, _KSUB * _PD), lambda i: (0, i)),
            pl.BlockSpec((_DIM, _KSUB * _PD), lambda i: (0, i)),
        ],
        out_specs=[
            pl.BlockSpec((_KSUB * _DIM, _PD), lambda i: (i, 0)),
            pl.BlockSpec((_KSUB * _DIM, _PD), lambda i: (i, 0)),
        ],
        out_shape=[
            jax.ShapeDtypeStruct((_PROWS, _PD), jnp.float32),
            jax.ShapeDtypeStruct((_PROWS, _PD), jnp.float32),
        ],
        compiler_params=pltpu.CompilerParams(
            dimension_semantics=("arbitrary",),
        ),
    )(t3, c3)

    mesh = plsc.VectorSubcoreMesh(core_axis_name="c", subcore_axis_name="s")
    k = pl.kernel(
        _dot_kernel,
        out_type=jax.ShapeDtypeStruct((_BATCH,), jnp.float32),
        mesh=mesh,
        scratch_types=[
            pltpu.VMEM((_BPW,), jnp.int32),
            pltpu.VMEM((_BPW,), jnp.int32),
            pltpu.VMEM((_BPW,), jnp.int32),
            pltpu.VMEM((_BPW,), jnp.int32),
            pltpu.VMEM((2, _CHUNK, _PD), jnp.float32),
            pltpu.VMEM((2, _CHUNK, _PD), jnp.float32),
            pltpu.VMEM((_BPW,), jnp.float32),
            pltpu.SemaphoreType.DMA,
            pltpu.SemaphoreType.DMA,
            pltpu.SemaphoreType.DMA,
            pltpu.SemaphoreType.DMA,
        ],
        compiler_params=pltpu.CompilerParams(
            needs_layout_passes=False,
        ),
    )
    return k(t, c, tconv, cconv).reshape(_BATCH, 1)


def kernel(target, context, target_table, context_table):
    return _run(target, context, target_table, context_table)
